# baseline (speedup 1.0000x reference)
import jax
import jax.numpy as jnp
from jax import lax
from jax.experimental import pallas as pl
from jax.experimental.pallas import tpu as pltpu
from jax.experimental.pallas import tpu_sc as plsc

B = 16384
D = 128
N = 100000
K = B // 2
NC = 2
NS = 16
NW = NC * NS
WIN = K // NW
PR = (N + D - 1) // D
SEG_ROWS = B // D // NW
TAIL_ROWS = 21
NG = B // 16


def _stat_tc(grad):
    def body(g_ref, o_ref):
        x = g_ref[...]
        ss = jnp.sum(x * x, axis=1)
        o_ref[...] = lax.bitcast_convert_type(
            jnp.sqrt(ss).reshape(16, 128), jnp.int32)

    return pl.pallas_call(
        body,
        grid=(8,),
        in_specs=[pl.BlockSpec((2048, 128), lambda i: (i, 0))],
        out_specs=pl.BlockSpec((16, 128), lambda i: (i, 0)),
        out_shape=jax.ShapeDtypeStruct((128, 128), jnp.int32),
    )(grad)


def _fused_body(bits_hbm, feats_hbm, emb_hbm, lid_hbm, pos_hbm, time_hbm,
                stat_v, hist2_v, hist_v, cum_v, cbuf_v, win_v, win2_v,
                rows_v, posb_v, timeb_v, filln_v, fillz_v, sem0, sem1):
    c = lax.axis_index("c")
    s = lax.axis_index("s")
    w = c * NS + s
    iota = lax.iota(jnp.int32, 16)
    ones = jnp.full((16,), 1, jnp.int32)
    zeros = jnp.full((16,), 0, jnp.int32)

    NBLK = 16
    GPB = NG // NBLK

    with jax.named_scope("ph_stage"):
        cps = pltpu.async_copy(bits_hbm, stat_v, sem0)

    with jax.named_scope("ph_fill"):
        @plsc.parallel_loop(0, 2048, unroll=8)
        def _zh(i):
            hist2_v[pl.ds(i * 16, 16)] = zeros

        neg1 = jnp.full((16,), -1, jnp.int32)
        for r in range(TAIL_ROWS):
            for j in range(8):
                filln_v[r, pl.ds(j * 16, 16)] = neg1
                fillz_v[r, pl.ds(j * 16, 16)] = zeros
        cps.wait()

    def _digit(nrows, k_rem):
        def _reduce(g, _nrows=nrows):
            acc = hist2_v[pl.ds(g * 16, 16)]
            hist2_v[pl.ds(g * 16, 16)] = zeros
            for r in range(1, _nrows):
                acc = acc + hist2_v[pl.ds(r * 256 + g * 16, 16)]
                hist2_v[pl.ds(r * 256 + g * 16, 16)] = zeros
            hist_v[pl.ds(g * 16, 16)] = acc
        plsc.parallel_loop(0, 16, unroll=2)(_reduce)

        def _cum(g, carry):
            v = hist_v[pl.ds(g * 16, 16)]
            cv = plsc.cumsum(v) + carry
            cum_v[pl.ds(g * 16, 16)] = cv
            return carry + jnp.sum(v)
        lax.fori_loop(0, 16, _cum, jnp.int32(0))

        def _bcnt(g, acc):
            cv = cum_v[pl.ds(g * 16, 16)]
            return acc + jnp.sum(jnp.where(cv < k_rem, 1, 0))
        b = lax.fori_loop(0, 16, _bcnt, jnp.int32(0))

        def _below(g, acc):
            cv = cum_v[pl.ds(g * 16, 16)]
            gl = g * 16 + iota
            return acc + jnp.sum(jnp.where(gl == b - 1, cv, 0))
        below = lax.fori_loop(0, 16, _below, jnp.int32(0))
        return b, below

    prefix = jnp.int32(0)
    k_rem = jnp.int32(K)

    with jax.named_scope("ph_radix0"):
        @plsc.parallel_loop(0, NG, unroll=8)
        def _hist0(i):
            v = stat_v[pl.ds(i * 16, 16)]
            b = lax.shift_right_logical(v, 24)
            reg = lax.rem(i, jnp.int32(8))
            plsc.addupdate_scatter(
                hist2_v, [(reg * 16 + iota) * 256 + b], ones)
        b, below = _digit(128, k_rem)
        k_rem = k_rem - below
        prefix = prefix | lax.shift_left(b, 24)

    with jax.named_scope("ph_radix1"):
        def _hist1(i):
            for half in range(2):
                v = stat_v[pl.ds((2 * i + half) * 16, 16)]
                m = (lax.shift_right_logical(v, 24) ==
                     lax.shift_right_logical(prefix, 24))
                b = lax.shift_right_logical(v, 16) & 0xFF
                plsc.addupdate_scatter(
                    hist2_v, [(iota + 16 * half) * 256 + b], ones, mask=m)
        plsc.parallel_loop(0, NG // 2, unroll=4)(_hist1)
        b, below = _digit(32, k_rem)
        k_rem = k_rem - below
        prefix = prefix | lax.shift_left(b, 16)

    with jax.named_scope("ph_radix2"):
        pref16 = lax.shift_right_logical(prefix, 16)
        mtot = []
        for bi in range(NBLK):
            def _mb(g, acc, _bi=bi):
                v = stat_v[pl.ds((_bi * GPB + g) * 16, 16)]
                m = lax.shift_right_logical(v, 16) == pref16
                return acc + jnp.where(m, 1, 0)
            acc = plsc.parallel_loop(0, GPB, unroll=4, carry=zeros)(_mb)
            mtot.append(jnp.sum(acc))
        moff = []
        cnt = jnp.int32(0)
        for bi in range(NBLK):
            moff.append(cnt)
            cnt = cnt + mtot[bi]

        for bi in range(NBLK):
            @pl.when(mtot[bi] > 0)
            def _cb(_bi=bi, _off0=moff[bi]):
                def _cc(g, off):
                    v = stat_v[pl.ds((_bi * GPB + g) * 16, 16)]
                    m = lax.shift_right_logical(v, 16) == pref16
                    plsc.store_compressed(cbuf_v.at[pl.ds(off, 16)], v,
                                          mask=m)
                    return off + jnp.sum(jnp.where(m, 1, 0))
                lax.fori_loop(0, GPB, _cc, _off0)
        cbuf_v[pl.ds(cnt, 16)] = jnp.full((16,), -1, jnp.int32)
        ngrc = lax.div(cnt + 15, jnp.int32(16))

        def _h2(i, _):
            v = cbuf_v[pl.ds(i * 16, 16)]
            m = lax.shift_right_logical(v, 16) == pref16
            b = lax.shift_right_logical(v, 8) & 0xFF
            plsc.addupdate_scatter(hist2_v, [iota * 256 + b], ones, mask=m)
            return 0
        lax.fori_loop(0, ngrc, _h2, 0)
        b, below = _digit(16, k_rem)
        k_rem = k_rem - below
        prefix = prefix | lax.shift_left(b, 8)

    with jax.named_scope("ph_radix3"):
        pref8 = lax.shift_right_logical(prefix, 8)

        def _hist3(i, _):
            v = cbuf_v[pl.ds(i * 16, 16)]
            m = lax.shift_right_logical(v, 8) == pref8
            b = v & 0xFF
            plsc.addupdate_scatter(hist2_v, [iota * 256 + b], ones, mask=m)
            return 0
        lax.fori_loop(0, ngrc, _hist3, 0)
        b, below = _digit(16, k_rem)
        k_rem = k_rem - below
        prefix = prefix | b

    t_bits = prefix

    own_g = w * (NG // NW)
    wlo = w * WIN
    with jax.named_scope("ph_scan"):
        btot = []
        for bi in range(NBLK):
            def _blk(g, acc, _bi=bi):
                v = stat_v[pl.ds((_bi * GPB + g) * 16, 16)]
                return acc + jnp.where(v <= t_bits, 1, 0)
            acc = plsc.parallel_loop(0, GPB, unroll=4, carry=zeros)(_blk)
            btot.append(jnp.sum(acc))

        bpre = []
        runp = jnp.int32(0)
        for bi in range(NBLK):
            bpre.append(runp)
            runp = runp + btot[bi]

        own_blk = w // 2
        pref_own = jnp.int32(0)
        for bi in range(NBLK):
            pref_own = pref_own + jnp.where(bi < own_blk, btot[bi], 0)

        def _pcnt(g, acc):
            v = stat_v[pl.ds(g * 16, 16)]
            return acc + jnp.sum(jnp.where(v <= t_bits, 1, 0))
        pref_own = lax.fori_loop(own_blk * GPB, own_g, _pcnt, pref_own)

        for bi in range(NBLK):
            hit = (bpre[bi] + btot[bi] > wlo) & (bpre[bi] < wlo + WIN)

            @pl.when(hit)
            def _rescan():
                def _scan(g, run):
                    v = stat_v[pl.ds((bi * GPB + g) * 16, 16)]
                    m = v <= t_bits
                    mi = jnp.where(m, 1, 0)
                    incl = plsc.cumsum(mi)
                    grank = run + incl - mi
                    inwin = m & (grank >= wlo) & (grank < wlo + WIN)
                    plsc.store_scatter(win_v, [grank - wlo],
                                       (bi * GPB + g) * 16 + iota,
                                       mask=inwin)
                    return run + jnp.sum(mi)
                lax.fori_loop(0, GPB, _scan, bpre[bi])

    with jax.named_scope("ph_out"):
        run2 = pref_own
        for g in range(32):
            v = stat_v[pl.ds(w * (B // NW) + g * 16, 16)]
            m = v <= t_bits
            mi = jnp.where(m, 1, 0)
            incl = plsc.cumsum(mi)
            grank = run2 + incl - mi
            run2 = run2 + jnp.sum(mi)
            sel = m & (grank < K)
            posb_v[g // 8, pl.ds((g % 8) * 16, 16)] = jnp.where(sel, grank, -1)
            timeb_v[g // 8, pl.ds((g % 8) * 16, 16)] = jnp.where(sel, 1, 0)

        pltpu.sync_copy(posb_v, pos_hbm.at[pl.ds(w * SEG_ROWS, SEG_ROWS)])
        pltpu.sync_copy(timeb_v, time_hbm.at[pl.ds(w * SEG_ROWS, SEG_ROWS)])

    with jax.named_scope("ph_tail"):
        tbase = jnp.minimum(B // D + w * TAIL_ROWS, PR - TAIL_ROWS)
        pltpu.sync_copy(filln_v, pos_hbm.at[pl.ds(tbase, TAIL_ROWS)])
        pltpu.sync_copy(fillz_v, time_hbm.at[pl.ds(tbase, TAIL_ROWS)])

    with jax.named_scope("ph_gather"):
        for r in range(2):
            for j in range(8):
                win2_v[r, pl.ds(j * 16, 16)] = win_v[pl.ds(r * 128 + j * 16, 16)]
        pltpu.sync_copy(win2_v, lid_hbm.at[pl.ds(2 * w, 2)])
        cp0 = pltpu.async_copy(feats_hbm.at[win2_v.at[0]], rows_v.at[0], sem0)
        cp1 = pltpu.async_copy(feats_hbm.at[win2_v.at[1]], rows_v.at[1], sem1)
        cp0.wait()
        pltpu.sync_copy(rows_v.at[0], emb_hbm.at[pl.ds(wlo, 128)])
        cp1.wait()
        pltpu.sync_copy(rows_v.at[1], emb_hbm.at[pl.ds(wlo + 128, 128)])


def _fused_sc(bits, feats):
    mesh = plsc.VectorSubcoreMesh(core_axis_name="c", subcore_axis_name="s",
                                  num_cores=NC, num_subcores=NS)
    return pl.kernel(
        _fused_body,
        out_type=(
            jax.ShapeDtypeStruct((K, D), jnp.float32),
            jax.ShapeDtypeStruct((K // D, D), jnp.int32),
            jax.ShapeDtypeStruct((PR, D), jnp.int32),
            jax.ShapeDtypeStruct((PR, D), jnp.int32),
        ),
        mesh=mesh,
        compiler_params=pltpu.CompilerParams(needs_layout_passes=False,
                                             use_tc_tiling_on_sc=False),
        scratch_types=[
            pltpu.VMEM((B,), jnp.int32),
            pltpu.VMEM((128 * 256,), jnp.int32),
            pltpu.VMEM((256,), jnp.int32),
            pltpu.VMEM((256,), jnp.int32),
            pltpu.VMEM((B + 16,), jnp.int32),
            pltpu.VMEM((WIN,), jnp.int32),
            pltpu.VMEM((2, 128), jnp.int32),
            pltpu.VMEM((2, 128, D), jnp.float32),
            pltpu.VMEM((SEG_ROWS, D), jnp.int32),
            pltpu.VMEM((SEG_ROWS, D), jnp.int32),
            pltpu.VMEM((TAIL_ROWS, D), jnp.int32),
            pltpu.VMEM((TAIL_ROWS, D), jnp.int32),
            pltpu.SemaphoreType.DMA,
            pltpu.SemaphoreType.DMA,
        ],
    )(bits, feats)


def kernel(gids, feats, grad, pos, time, emb, index_to_gid):
    bits = _stat_tc(grad).reshape(B)
    emb_o, lid2, pos2, time2 = _fused_sc(bits, feats)
    return (emb_o, pos2.reshape(-1)[:N], lid2.reshape(-1),
            time2.reshape(-1)[:N])


# speedup vs baseline: 1.4450x; 1.4450x over previous
import jax
import jax.numpy as jnp
from jax import lax
from jax.experimental import pallas as pl
from jax.experimental.pallas import tpu as pltpu
from jax.experimental.pallas import tpu_sc as plsc

B = 16384
D = 128
N = 100000
K = B // 2
NC = 2
NS = 16
NW = NC * NS
WIN = K // NW
PR = (N + D - 1) // D
SEG_ROWS = B // D // NW
TAIL_ROWS = 21
NG = B // 16


def _stat_tc(grad):
    def body(g_ref, o_ref):
        x = g_ref[...]
        ss = jnp.sum(x * x, axis=1)
        o_ref[...] = lax.bitcast_convert_type(
            jnp.sqrt(ss).reshape(16, 128), jnp.int32)

    return pl.pallas_call(
        body,
        grid=(8,),
        in_specs=[pl.BlockSpec((2048, 128), lambda i: (i, 0))],
        out_specs=pl.BlockSpec((16, 128), lambda i: (i, 0)),
        out_shape=jax.ShapeDtypeStruct((128, 128), jnp.int32),
    )(grad)


def _fused_body(bits_hbm, feats_hbm, emb_hbm, lid_hbm, pos_hbm, time_hbm,
                stat_v, hist2_v, hist_v, cum_v, cbuf_v, win_v, win2_v,
                rows_v, posb_v, timeb_v, filln_v, fillz_v, sem0, sem1):
    c = lax.axis_index("c")
    s = lax.axis_index("s")
    w = c * NS + s
    iota = lax.iota(jnp.int32, 16)
    ones = jnp.full((16,), 1, jnp.int32)
    zeros = jnp.full((16,), 0, jnp.int32)

    NBLK = 16
    GPB = NG // NBLK

    with jax.named_scope("ph_stage"):
        cps = pltpu.async_copy(bits_hbm, stat_v, sem0)

    with jax.named_scope("ph_fill"):
        @plsc.parallel_loop(0, 512, unroll=8)
        def _zh(i):
            hist2_v[pl.ds(i * 16, 16)] = zeros

        neg1 = jnp.full((16,), -1, jnp.int32)
        for r in range(TAIL_ROWS):
            for j in range(8):
                filln_v[r, pl.ds(j * 16, 16)] = neg1
                fillz_v[r, pl.ds(j * 16, 16)] = zeros
        cps.wait()

    def _digit(nrows, k_rem):
        def _reduce(g, _nrows=nrows):
            acc = hist2_v[pl.ds(g * 16, 16)]
            hist2_v[pl.ds(g * 16, 16)] = zeros
            for r in range(1, _nrows):
                acc = acc + hist2_v[pl.ds(r * 256 + g * 16, 16)]
                hist2_v[pl.ds(r * 256 + g * 16, 16)] = zeros
            hist_v[pl.ds(g * 16, 16)] = acc
        plsc.parallel_loop(0, 16, unroll=2)(_reduce)

        def _cum(g, carry):
            v = hist_v[pl.ds(g * 16, 16)]
            cv = plsc.cumsum(v) + carry
            cum_v[pl.ds(g * 16, 16)] = cv
            return carry + jnp.sum(v)
        lax.fori_loop(0, 16, _cum, jnp.int32(0))

        def _bcnt(g, acc):
            cv = cum_v[pl.ds(g * 16, 16)]
            return acc + jnp.sum(jnp.where(cv < k_rem, 1, 0))
        b = lax.fori_loop(0, 16, _bcnt, jnp.int32(0))

        def _below(g, acc):
            cv = cum_v[pl.ds(g * 16, 16)]
            gl = g * 16 + iota
            return acc + jnp.sum(jnp.where(gl == b - 1, cv, 0))
        below = lax.fori_loop(0, 16, _below, jnp.int32(0))
        return b, below

    prefix = jnp.int32(0)
    k_rem = jnp.int32(K)

    with jax.named_scope("ph_radix0"):
        @plsc.parallel_loop(0, NG, unroll=8)
        def _hist0(i):
            v = stat_v[pl.ds(i * 16, 16)]
            b = lax.shift_right_logical(v, 24)
            cnts, last = plsc.scan_count(b)
            reg = lax.rem(i, jnp.int32(8))
            plsc.addupdate_scatter(hist2_v, [reg * 256 + b], cnts, mask=last)
        b, below = _digit(8, k_rem)
        k_rem = k_rem - below
        prefix = prefix | lax.shift_left(b, 24)

    with jax.named_scope("ph_radix1"):
        def _hist1(i):
            for half in range(2):
                v = stat_v[pl.ds((2 * i + half) * 16, 16)]
                m = (lax.shift_right_logical(v, 24) ==
                     lax.shift_right_logical(prefix, 24))
                b = lax.shift_right_logical(v, 16) & 0xFF
                plsc.addupdate_scatter(
                    hist2_v, [(iota + 16 * half) * 256 + b], ones, mask=m)
        plsc.parallel_loop(0, NG // 2, unroll=4)(_hist1)
        b, below = _digit(32, k_rem)
        k_rem = k_rem - below
        prefix = prefix | lax.shift_left(b, 16)

    QN = 4
    QSTRIDE = B // QN + 16
    with jax.named_scope("ph_radix2"):
        pref16 = lax.shift_right_logical(prefix, 16)

        def _cc(i, offs):
            new = []
            for q in range(QN):
                off = offs[q]
                v = stat_v[pl.ds((i * QN + q) * 16, 16)]
                m = lax.shift_right_logical(v, 16) == pref16
                plsc.store_compressed(cbuf_v.at[pl.ds(off, 16)], v, mask=m)
                new.append(off + jnp.sum(jnp.where(m, 1, 0)))
            return tuple(new)
        offs = plsc.parallel_loop(
            0, NG // QN, unroll=2,
            carry=tuple(jnp.int32(q * QSTRIDE) for q in range(QN)))(_cc)
        sent = jnp.full((16,), -1, jnp.int32)
        ngrc = []
        for q in range(QN):
            cbuf_v[pl.ds(offs[q], 16)] = sent
            ngrc.append(lax.div(offs[q] - q * QSTRIDE + 15, jnp.int32(16)))

        def _h2(i, _, _q=None):
            v = cbuf_v[pl.ds(_q * QSTRIDE + i * 16, 16)]
            m = lax.shift_right_logical(v, 16) == pref16
            b = lax.shift_right_logical(v, 8) & 0xFF
            plsc.addupdate_scatter(hist2_v, [iota * 256 + b], ones, mask=m)
            return 0
        for q in range(QN):
            lax.fori_loop(0, ngrc[q],
                          lambda i, a, _q=q: _h2(i, a, _q=_q), 0)
        b, below = _digit(16, k_rem)
        k_rem = k_rem - below
        prefix = prefix | lax.shift_left(b, 8)

    with jax.named_scope("ph_radix3"):
        pref8 = lax.shift_right_logical(prefix, 8)

        def _hist3(i, _, _q=None):
            v = cbuf_v[pl.ds(_q * QSTRIDE + i * 16, 16)]
            m = lax.shift_right_logical(v, 8) == pref8
            b = v & 0xFF
            plsc.addupdate_scatter(hist2_v, [iota * 256 + b], ones, mask=m)
            return 0
        for q in range(QN):
            lax.fori_loop(0, ngrc[q],
                          lambda i, a, _q=q: _hist3(i, a, _q=_q), 0)
        b, below = _digit(16, k_rem)
        k_rem = k_rem - below
        prefix = prefix | b

    t_bits = prefix

    own_g = w * (NG // NW)
    wlo = w * WIN
    with jax.named_scope("ph_scan"):
        btot = []
        for bi in range(NBLK):
            def _blk(g, acc, _bi=bi):
                v = stat_v[pl.ds((_bi * GPB + g) * 16, 16)]
                return acc + jnp.where(v <= t_bits, 1, 0)
            acc = plsc.parallel_loop(0, GPB, unroll=4, carry=zeros)(_blk)
            btot.append(jnp.sum(acc))

        bpre = []
        runp = jnp.int32(0)
        for bi in range(NBLK):
            bpre.append(runp)
            runp = runp + btot[bi]

        own_blk = w // 2
        pref_own = jnp.int32(0)
        for bi in range(NBLK):
            pref_own = pref_own + jnp.where(bi < own_blk, btot[bi], 0)

        def _pcnt(g, acc):
            v = stat_v[pl.ds(g * 16, 16)]
            return acc + jnp.sum(jnp.where(v <= t_bits, 1, 0))
        pref_own = lax.fori_loop(own_blk * GPB, own_g, _pcnt, pref_own)

        for bi in range(NBLK):
            hit = (bpre[bi] + btot[bi] > wlo) & (bpre[bi] < wlo + WIN)

            @pl.when(hit)
            def _rescan():
                def _scan(g, run):
                    v = stat_v[pl.ds((bi * GPB + g) * 16, 16)]
                    m = v <= t_bits
                    mi = jnp.where(m, 1, 0)
                    incl = plsc.cumsum(mi)
                    grank = run + incl - mi
                    inwin = m & (grank >= wlo) & (grank < wlo + WIN)
                    plsc.store_scatter(win_v, [grank - wlo],
                                       (bi * GPB + g) * 16 + iota,
                                       mask=inwin)
                    return run + jnp.sum(mi)
                lax.fori_loop(0, GPB, _scan, bpre[bi])

    with jax.named_scope("ph_out"):
        run2 = pref_own
        for g in range(32):
            v = stat_v[pl.ds(w * (B // NW) + g * 16, 16)]
            m = v <= t_bits
            mi = jnp.where(m, 1, 0)
            incl = plsc.cumsum(mi)
            grank = run2 + incl - mi
            run2 = run2 + jnp.sum(mi)
            sel = m & (grank < K)
            posb_v[g // 8, pl.ds((g % 8) * 16, 16)] = jnp.where(sel, grank, -1)
            timeb_v[g // 8, pl.ds((g % 8) * 16, 16)] = jnp.where(sel, 1, 0)

        pltpu.sync_copy(posb_v, pos_hbm.at[pl.ds(w * SEG_ROWS, SEG_ROWS)])
        pltpu.sync_copy(timeb_v, time_hbm.at[pl.ds(w * SEG_ROWS, SEG_ROWS)])

    with jax.named_scope("ph_tail"):
        tbase = jnp.minimum(B // D + w * TAIL_ROWS, PR - TAIL_ROWS)
        pltpu.sync_copy(filln_v, pos_hbm.at[pl.ds(tbase, TAIL_ROWS)])
        pltpu.sync_copy(fillz_v, time_hbm.at[pl.ds(tbase, TAIL_ROWS)])

    with jax.named_scope("ph_gather"):
        for r in range(2):
            for j in range(8):
                win2_v[r, pl.ds(j * 16, 16)] = win_v[pl.ds(r * 128 + j * 16, 16)]
        pltpu.sync_copy(win2_v, lid_hbm.at[pl.ds(2 * w, 2)])
        cp0 = pltpu.async_copy(feats_hbm.at[win2_v.at[0]], rows_v.at[0], sem0)
        cp1 = pltpu.async_copy(feats_hbm.at[win2_v.at[1]], rows_v.at[1], sem1)
        cp0.wait()
        pltpu.sync_copy(rows_v.at[0], emb_hbm.at[pl.ds(wlo, 128)])
        cp1.wait()
        pltpu.sync_copy(rows_v.at[1], emb_hbm.at[pl.ds(wlo + 128, 128)])


def _fused_sc(bits, feats):
    mesh = plsc.VectorSubcoreMesh(core_axis_name="c", subcore_axis_name="s",
                                  num_cores=NC, num_subcores=NS)
    return pl.kernel(
        _fused_body,
        out_type=(
            jax.ShapeDtypeStruct((K, D), jnp.float32),
            jax.ShapeDtypeStruct((K // D, D), jnp.int32),
            jax.ShapeDtypeStruct((PR, D), jnp.int32),
            jax.ShapeDtypeStruct((PR, D), jnp.int32),
        ),
        mesh=mesh,
        compiler_params=pltpu.CompilerParams(needs_layout_passes=False,
                                             use_tc_tiling_on_sc=False),
        scratch_types=[
            pltpu.VMEM((B,), jnp.int32),
            pltpu.VMEM((32 * 256,), jnp.int32),
            pltpu.VMEM((256,), jnp.int32),
            pltpu.VMEM((256,), jnp.int32),
            pltpu.VMEM((4 * (B // 4 + 16),), jnp.int32),
            pltpu.VMEM((WIN,), jnp.int32),
            pltpu.VMEM((2, 128), jnp.int32),
            pltpu.VMEM((2, 128, D), jnp.float32),
            pltpu.VMEM((SEG_ROWS, D), jnp.int32),
            pltpu.VMEM((SEG_ROWS, D), jnp.int32),
            pltpu.VMEM((TAIL_ROWS, D), jnp.int32),
            pltpu.VMEM((TAIL_ROWS, D), jnp.int32),
            pltpu.SemaphoreType.DMA,
            pltpu.SemaphoreType.DMA,
        ],
    )(bits, feats)


def kernel(gids, feats, grad, pos, time, emb, index_to_gid):
    bits = _stat_tc(grad).reshape(B)
    emb_o, lid2, pos2, time2 = _fused_sc(bits, feats)
    return (emb_o, pos2.reshape(-1)[:N], lid2.reshape(-1),
            time2.reshape(-1)[:N])
